# baseline (speedup 1.0000x reference)
import functools

import jax
import jax.numpy as jnp
from jax import lax
from jax.experimental import pallas as pl
from jax.experimental.pallas import tpu as pltpu
from jax.experimental.pallas import tpu_sc as plsc

_N = 16384
_C = 340
_NB_SC = 8192
_NC = 2
_NS = 16
_L = 16
_NW = _NC * _NS
_RPW = _NB_SC // _NW
_CK = 64
_NCK = _C // _CK
_REM = _C - _NCK * _CK
_NG = _RPW // _L
_TCB = 1024


def _tree_sum(vs):
    vs = list(vs)
    while len(vs) > 1:
        nxt = [vs[i] + vs[i + 1] for i in range(0, len(vs) - 1, 2)]
        if len(vs) % 2:
            nxt.append(vs[-1])
        vs = nxt
    return vs[0]


def _sc_body(x_hbm, t_hbm, out_hbm, xb0, xb1, xbr, tvb, rsb, xeb, ov,
             sem0, sem1):
    wid = lax.axis_index("s") * _NC + lax.axis_index("c")
    base = wid * _RPW
    lane = lax.broadcasted_iota(jnp.int32, (_L,), 0)

    pltpu.sync_copy(t_hbm.at[pl.ds(base, _RPW)], tvb)

    zero = jnp.zeros((_L,), jnp.float32)

    def zero_body(g, c):
        rsb[pl.ds(g * _L, _L)] = zero
        xeb[pl.ds(g * _L, _L)] = zero
        return c

    lax.fori_loop(0, _NG, zero_body, 0, unroll=False)

    def chunk_src(s, nrows=_CK):
        return x_hbm.at[pl.ds(s * _CK, nrows), pl.ds(base, _RPW)]

    def compute(buf, s, nrows):
        def gbody(g, c):
            col = g * _L
            tl = tvb[pl.ds(col, _L)] - s * _CK
            evs = []
            for jr in range(nrows):
                evs.append(jnp.exp(buf[jr, pl.ds(col, _L)]))
            rsb[pl.ds(col, _L)] += _tree_sum(evs)
            m = (tl >= 0) & (tl < nrows)
            tr = jnp.clip(tl, 0, nrows - 1)
            xg = plsc.load_gather(buf, [tr, col + lane])
            xeb[pl.ds(col, _L)] += jnp.where(m, jnp.exp(xg), 0.0)
            return c

        lax.fori_loop(0, _NG, gbody, 0, unroll=False)

    pltpu.async_copy(chunk_src(0), xb0, sem0)

    def pair_body(p, c):
        s0 = p * 2
        s1 = s0 + 1
        pltpu.async_copy(chunk_src(s1), xb1, sem1)
        pltpu.make_async_copy(chunk_src(s0), xb0, sem0).wait()
        compute(xb0, s0, _CK)
        pltpu.async_copy(chunk_src(s0 + 2), xb0, sem0)
        pltpu.make_async_copy(chunk_src(s1), xb1, sem1).wait()
        compute(xb1, s1, _CK)
        return c

    lax.fori_loop(0, (_NCK - 1) // 2, pair_body, 0, unroll=False)

    pltpu.make_async_copy(chunk_src(_NCK - 1), xb0, sem0).wait()
    compute(xb0, _NCK - 1, _CK)

    pltpu.sync_copy(chunk_src(_NCK, _REM), xbr)
    compute(xbr, _NCK, _REM)

    total = jnp.zeros((_L,), jnp.float32)
    for g in range(_NG):
        rs = rsb[pl.ds(g * _L, _L)]
        xe = xeb[pl.ds(g * _L, _L)]
        total = total + (rs - xe) / xe
    ov[...] = total
    pltpu.sync_copy(ov, out_hbm.at[wid])


def _tc_body(x_ref, t_ref, o_ref):
    i = pl.program_id(0)
    x = x_ref[...]
    t = t_ref[...]
    ex = jnp.exp(x)
    rowsum = jnp.sum(ex, axis=0)
    cls = lax.broadcasted_iota(jnp.int32, (_C, _TCB), 0)
    xt = jnp.sum(jnp.where(cls == t, x, 0.0), axis=0)
    part = jnp.sum(jnp.exp(-xt) * (rowsum - jnp.exp(xt)))

    @pl.when(i == 0)
    def _():
        o_ref[0, 0] = 0.0

    o_ref[0, 0] += part


@jax.jit
def kernel(input, target):
    xt = input.T
    t32 = target.astype(jnp.int32)
    run_sc = functools.partial(
        pl.kernel,
        mesh=plsc.VectorSubcoreMesh(core_axis_name="c", subcore_axis_name="s"),
        compiler_params=pltpu.CompilerParams(needs_layout_passes=False),
        out_type=jax.ShapeDtypeStruct((_NW, _L), jnp.float32),
        scratch_types=[
            pltpu.VMEM((_CK, _RPW), jnp.float32),
            pltpu.VMEM((_CK, _RPW), jnp.float32),
            pltpu.VMEM((_REM, _RPW), jnp.float32),
            pltpu.VMEM((_RPW,), jnp.int32),
            pltpu.VMEM((_RPW,), jnp.float32),
            pltpu.VMEM((_RPW,), jnp.float32),
            pltpu.VMEM((_L,), jnp.float32),
            pltpu.SemaphoreType.DMA,
            pltpu.SemaphoreType.DMA,
        ],
    )(_sc_body)
    sc_parts = run_sc(xt, t32)

    ntc = (_N - _NB_SC) // _TCB
    tc_part = pl.pallas_call(
        _tc_body,
        grid=(ntc,),
        in_specs=[
            pl.BlockSpec((_C, _TCB), lambda i: (0, _NB_SC // _TCB + i)),
            pl.BlockSpec((1, _TCB), lambda i: (0, _NB_SC // _TCB + i)),
        ],
        out_specs=pl.BlockSpec(
            (1, 1), lambda i: (0, 0), memory_space=pltpu.SMEM
        ),
        out_shape=jax.ShapeDtypeStruct((1, 1), jnp.float32),
    )(xt, t32.reshape(1, _N))

    return jnp.log(1.0 + (jnp.sum(sc_parts) + tc_part[0, 0]))


# speedup vs baseline: 1.0984x; 1.0984x over previous
import functools

import jax
import jax.numpy as jnp
from jax import lax
from jax.experimental import pallas as pl
from jax.experimental.pallas import tpu as pltpu
from jax.experimental.pallas import tpu_sc as plsc

_N = 16384
_C = 340
_NB_SC = 4096
_NC = 2
_NS = 16
_L = 16
_NW = _NC * _NS
_RPW = _NB_SC // _NW
_CK = 64
_NCK = _C // _CK
_REM = _C - _NCK * _CK
_NG = _RPW // _L
_TCB = 1024


def _tree_sum(vs):
    vs = list(vs)
    while len(vs) > 1:
        nxt = [vs[i] + vs[i + 1] for i in range(0, len(vs) - 1, 2)]
        if len(vs) % 2:
            nxt.append(vs[-1])
        vs = nxt
    return vs[0]


def _sc_body(x_hbm, t_hbm, out_hbm, xb0, xb1, xbr, tvb, rsb, xeb, ov,
             sem0, sem1):
    wid = lax.axis_index("s") * _NC + lax.axis_index("c")
    base = wid * _RPW
    lane = lax.broadcasted_iota(jnp.int32, (_L,), 0)

    pltpu.sync_copy(t_hbm.at[pl.ds(base, _RPW)], tvb)

    zero = jnp.zeros((_L,), jnp.float32)

    def zero_body(g, c):
        rsb[pl.ds(g * _L, _L)] = zero
        xeb[pl.ds(g * _L, _L)] = zero
        return c

    lax.fori_loop(0, _NG, zero_body, 0, unroll=False)

    def chunk_src(s, nrows=_CK):
        return x_hbm.at[pl.ds(s * _CK, nrows), pl.ds(base, _RPW)]

    def compute(buf, s, nrows):
        def gbody(g, c):
            col = g * _L
            tl = tvb[pl.ds(col, _L)] - s * _CK
            evs = []
            for jr in range(nrows):
                evs.append(jnp.exp(buf[jr, pl.ds(col, _L)]))
            rsb[pl.ds(col, _L)] += _tree_sum(evs)
            m = (tl >= 0) & (tl < nrows)
            tr = jnp.clip(tl, 0, nrows - 1)
            xg = plsc.load_gather(buf, [tr, col + lane])
            xeb[pl.ds(col, _L)] += jnp.where(m, jnp.exp(xg), 0.0)
            return c

        lax.fori_loop(0, _NG, gbody, 0, unroll=False)

    pltpu.async_copy(chunk_src(0), xb0, sem0)

    def pair_body(p, c):
        s0 = p * 2
        s1 = s0 + 1
        pltpu.async_copy(chunk_src(s1), xb1, sem1)
        pltpu.make_async_copy(chunk_src(s0), xb0, sem0).wait()
        compute(xb0, s0, _CK)
        pltpu.async_copy(chunk_src(s0 + 2), xb0, sem0)
        pltpu.make_async_copy(chunk_src(s1), xb1, sem1).wait()
        compute(xb1, s1, _CK)
        return c

    lax.fori_loop(0, (_NCK - 1) // 2, pair_body, 0, unroll=False)

    pltpu.make_async_copy(chunk_src(_NCK - 1), xb0, sem0).wait()
    compute(xb0, _NCK - 1, _CK)

    pltpu.sync_copy(chunk_src(_NCK, _REM), xbr)
    compute(xbr, _NCK, _REM)

    total = jnp.zeros((_L,), jnp.float32)
    for g in range(_NG):
        rs = rsb[pl.ds(g * _L, _L)]
        xe = xeb[pl.ds(g * _L, _L)]
        total = total + (rs - xe) / xe
    ov[...] = total
    pltpu.sync_copy(ov, out_hbm.at[wid])


def _tc_body(x_ref, t_ref, o_ref):
    i = pl.program_id(0)
    x = x_ref[...]
    t = t_ref[...]
    ex = jnp.exp(x)
    rowsum = jnp.sum(ex, axis=0)
    cls = lax.broadcasted_iota(jnp.int32, (_C, _TCB), 0)
    xt = jnp.sum(jnp.where(cls == t, x, 0.0), axis=0)
    part = jnp.sum(jnp.exp(-xt) * (rowsum - jnp.exp(xt)))

    @pl.when(i == 0)
    def _():
        o_ref[0, 0] = 0.0

    o_ref[0, 0] += part


@jax.jit
def kernel(input, target):
    xt = input.T
    t32 = target.astype(jnp.int32)
    run_sc = functools.partial(
        pl.kernel,
        mesh=plsc.VectorSubcoreMesh(core_axis_name="c", subcore_axis_name="s"),
        compiler_params=pltpu.CompilerParams(needs_layout_passes=False),
        out_type=jax.ShapeDtypeStruct((_NW, _L), jnp.float32),
        scratch_types=[
            pltpu.VMEM((_CK, _RPW), jnp.float32),
            pltpu.VMEM((_CK, _RPW), jnp.float32),
            pltpu.VMEM((_REM, _RPW), jnp.float32),
            pltpu.VMEM((_RPW,), jnp.int32),
            pltpu.VMEM((_RPW,), jnp.float32),
            pltpu.VMEM((_RPW,), jnp.float32),
            pltpu.VMEM((_L,), jnp.float32),
            pltpu.SemaphoreType.DMA,
            pltpu.SemaphoreType.DMA,
        ],
    )(_sc_body)
    sc_parts = run_sc(xt, t32)

    ntc = (_N - _NB_SC) // _TCB
    tc_part = pl.pallas_call(
        _tc_body,
        grid=(ntc,),
        in_specs=[
            pl.BlockSpec((_C, _TCB), lambda i: (0, _NB_SC // _TCB + i)),
            pl.BlockSpec((1, _TCB), lambda i: (0, _NB_SC // _TCB + i)),
        ],
        out_specs=pl.BlockSpec(
            (1, 1), lambda i: (0, 0), memory_space=pltpu.SMEM
        ),
        out_shape=jax.ShapeDtypeStruct((1, 1), jnp.float32),
    )(xt, t32.reshape(1, _N))

    return jnp.log(1.0 + (jnp.sum(sc_parts) + tc_part[0, 0]))
